# Initial kernel scaffold; baseline (speedup 1.0000x reference)
import jax
import jax.numpy as jnp
from jax.experimental import pallas as pl


def kernel(current_genes, ig):
    raise NotImplementedError("write your pallas kernel here")


# speedup vs baseline: 4.3841x; 4.3841x over previous
import functools

import jax
import jax.numpy as jnp
from jax import lax
from jax.experimental import pallas as pl
from jax.experimental.pallas import tpu as pltpu
from jax.experimental.pallas import tpu_sc as plsc

VOCAB = 1000
VOCAB_PAD = 1008
BATCH = 16384
NUM_CORES = 2
NUM_SUBCORES = 16
LANES = 16
NUM_WORKERS = NUM_CORES * NUM_SUBCORES
B_PER_W = BATCH // NUM_WORKERS

_mesh = plsc.VectorSubcoreMesh(core_axis_name="c", subcore_axis_name="s")


@functools.partial(
    pl.kernel,
    mesh=_mesh,
    out_type=jax.ShapeDtypeStruct((BATCH,), jnp.float32),
    scratch_types=[
        pltpu.VMEM((VOCAB_PAD,), jnp.float32),
        pltpu.VMEM((B_PER_W,), jnp.int32),
        pltpu.VMEM((B_PER_W,), jnp.float32),
    ],
    compiler_params=pltpu.CompilerParams(needs_layout_passes=False),
)
def _ig_gather_sigmoid(genes_hbm, ig_hbm, out_hbm, tab_v, idx_v, out_v):
    wid = lax.axis_index("s") * NUM_CORES + lax.axis_index("c")
    base = wid * B_PER_W
    pltpu.sync_copy(ig_hbm, tab_v)
    pltpu.sync_copy(genes_hbm.at[pl.ds(base, B_PER_W)], idx_v)
    for i in range(B_PER_W // LANES):
        idx = idx_v[pl.ds(i * LANES, LANES)]
        g = plsc.load_gather(tab_v, [idx])
        out_v[pl.ds(i * LANES, LANES)] = 1.0 / (1.0 + jnp.exp(-g))
    pltpu.sync_copy(out_v, out_hbm.at[pl.ds(base, B_PER_W)])


def kernel(current_genes, ig):
    ig_pad = jnp.pad(ig, (0, VOCAB_PAD - VOCAB))
    return _ig_gather_sigmoid(current_genes.astype(jnp.int32), ig_pad)


# speedup vs baseline: 4.5016x; 1.0268x over previous
import functools

import jax
import jax.numpy as jnp
from jax import lax
from jax.experimental import pallas as pl
from jax.experimental.pallas import tpu as pltpu
from jax.experimental.pallas import tpu_sc as plsc

VOCAB = 1000
VOCAB_PAD = 1008
BATCH = 16384
NUM_CORES = 2
NUM_SUBCORES = 16
LANES = 16
NUM_WORKERS = NUM_CORES * NUM_SUBCORES
B_PER_W = BATCH // NUM_WORKERS

_mesh = plsc.VectorSubcoreMesh(core_axis_name="c", subcore_axis_name="s")


@functools.partial(
    pl.kernel,
    mesh=_mesh,
    out_type=jax.ShapeDtypeStruct((BATCH,), jnp.float32),
    scratch_types=[
        pltpu.VMEM((VOCAB,), jnp.float32),
        pltpu.VMEM((B_PER_W,), jnp.int32),
        pltpu.VMEM((B_PER_W,), jnp.float32),
        pltpu.SemaphoreType.DMA,
        pltpu.SemaphoreType.DMA,
    ],
    compiler_params=pltpu.CompilerParams(needs_layout_passes=False),
)
def _ig_gather_sigmoid(genes_hbm, ig_hbm, out_hbm, tab_v, idx_v, out_v,
                       sem_tab, sem_idx):
    wid = lax.axis_index("s") * NUM_CORES + lax.axis_index("c")
    base = wid * B_PER_W
    tab_cp = pltpu.async_copy(ig_hbm, tab_v, sem_tab)
    idx_cp = pltpu.async_copy(genes_hbm.at[pl.ds(base, B_PER_W)], idx_v,
                              sem_idx)
    tab_cp.wait()
    idx_cp.wait()
    for i in range(B_PER_W // LANES):
        idx = idx_v[pl.ds(i * LANES, LANES)]
        g = plsc.load_gather(tab_v, [idx])
        out_v[pl.ds(i * LANES, LANES)] = 1.0 / (1.0 + jnp.exp(-g))
    pltpu.sync_copy(out_v, out_hbm.at[pl.ds(base, B_PER_W)])


def kernel(current_genes, ig):
    return _ig_gather_sigmoid(current_genes.astype(jnp.int32), ig)


# speedup vs baseline: 4.5069x; 1.0012x over previous
import functools

import jax
import jax.numpy as jnp
from jax import lax
from jax.experimental import pallas as pl
from jax.experimental.pallas import tpu as pltpu
from jax.experimental.pallas import tpu_sc as plsc

VOCAB = 1000
VOCAB_PAD = 1008
BATCH = 16384
NUM_CORES = 2
NUM_SUBCORES = 16
LANES = 16
NUM_WORKERS = NUM_CORES * NUM_SUBCORES
B_PER_W = BATCH // NUM_WORKERS

_mesh = plsc.VectorSubcoreMesh(core_axis_name="c", subcore_axis_name="s")


@functools.partial(
    pl.kernel,
    mesh=_mesh,
    out_type=jax.ShapeDtypeStruct((BATCH,), jnp.float32),
    scratch_types=[
        pltpu.VMEM((VOCAB,), jnp.float32),
        pltpu.VMEM((B_PER_W,), jnp.int32),
        pltpu.VMEM((B_PER_W,), jnp.float32),
        pltpu.SemaphoreType.DMA,
        pltpu.SemaphoreType.DMA,
        pltpu.SemaphoreType.DMA,
    ],
    compiler_params=pltpu.CompilerParams(needs_layout_passes=False),
)
def _ig_gather_sigmoid(genes_hbm, ig_hbm, out_hbm, tab_v, idx_v, out_v,
                       sem_tab, sem_idx, sem_out):
    wid = lax.axis_index("s") * NUM_CORES + lax.axis_index("c")
    base = wid * B_PER_W
    tab_cp = pltpu.async_copy(ig_hbm, tab_v, sem_tab)
    idx_cp = pltpu.async_copy(genes_hbm.at[pl.ds(base, B_PER_W)], idx_v,
                              sem_idx)
    tab_cp.wait()
    idx_cp.wait()
    half = B_PER_W // 2
    for i in range(half // LANES):
        idx = idx_v[pl.ds(i * LANES, LANES)]
        g = plsc.load_gather(tab_v, [idx])
        out_v[pl.ds(i * LANES, LANES)] = 1.0 / (1.0 + jnp.exp(-g))
    out0_cp = pltpu.async_copy(out_v.at[pl.ds(0, half)],
                               out_hbm.at[pl.ds(base, half)], sem_out)
    for i in range(half // LANES, B_PER_W // LANES):
        idx = idx_v[pl.ds(i * LANES, LANES)]
        g = plsc.load_gather(tab_v, [idx])
        out_v[pl.ds(i * LANES, LANES)] = 1.0 / (1.0 + jnp.exp(-g))
    out1_cp = pltpu.async_copy(out_v.at[pl.ds(half, half)],
                               out_hbm.at[pl.ds(base + half, half)], sem_out)
    out0_cp.wait()
    out1_cp.wait()


def kernel(current_genes, ig):
    return _ig_gather_sigmoid(current_genes.astype(jnp.int32), ig)


# speedup vs baseline: 4.6221x; 1.0256x over previous
import functools

import jax
import jax.numpy as jnp
from jax import lax
from jax.experimental import pallas as pl
from jax.experimental.pallas import tpu as pltpu
from jax.experimental.pallas import tpu_sc as plsc

VOCAB = 1000
VOCAB_PAD = 1008
BATCH = 16384
NUM_CORES = 2
NUM_SUBCORES = 16
LANES = 16
NUM_WORKERS = NUM_CORES * NUM_SUBCORES
B_PER_W = BATCH // NUM_WORKERS

_mesh = plsc.VectorSubcoreMesh(core_axis_name="c", subcore_axis_name="s")


@functools.partial(
    pl.kernel,
    mesh=_mesh,
    out_type=jax.ShapeDtypeStruct((BATCH,), jnp.float32),
    scratch_types=[
        pltpu.VMEM((VOCAB,), jnp.float32),
        pltpu.VMEM((B_PER_W,), jnp.int32),
        pltpu.VMEM((B_PER_W,), jnp.float32),
        pltpu.SemaphoreType.DMA,
        pltpu.SemaphoreType.DMA,
        pltpu.SemaphoreType.DMA,
    ],
    compiler_params=pltpu.CompilerParams(needs_layout_passes=False),
)
def _ig_gather_sigmoid(genes_hbm, ig_hbm, out_hbm, tab_v, idx_v, out_v,
                       sem_tab, sem_idx, sem_out):
    wid = lax.axis_index("s") * NUM_CORES + lax.axis_index("c")
    base = wid * B_PER_W
    tab_cp = pltpu.async_copy(ig_hbm, tab_v, sem_tab)
    idx_cp = pltpu.async_copy(genes_hbm.at[pl.ds(base, B_PER_W)], idx_v,
                              sem_idx)
    tab_cp.wait()
    idx_cp.wait()
    half = B_PER_W // 2

    def step(i, _):
        off = pl.multiple_of(i * LANES, LANES)
        idx = idx_v[pl.ds(off, LANES)]
        g = plsc.load_gather(tab_v, [idx])
        out_v[pl.ds(off, LANES)] = 1.0 / (1.0 + jnp.exp(-g))
        return _

    lax.fori_loop(0, half // LANES, step, 0, unroll=1)
    out0_cp = pltpu.async_copy(out_v.at[pl.ds(0, half)],
                               out_hbm.at[pl.ds(base, half)], sem_out)
    lax.fori_loop(half // LANES, B_PER_W // LANES, step, 0, unroll=1)
    out1_cp = pltpu.async_copy(out_v.at[pl.ds(half, half)],
                               out_hbm.at[pl.ds(base + half, half)], sem_out)
    out0_cp.wait()
    out1_cp.wait()


def kernel(current_genes, ig):
    return _ig_gather_sigmoid(current_genes.astype(jnp.int32), ig)


# speedup vs baseline: 5.1217x; 1.1081x over previous
import functools

import jax
import jax.numpy as jnp
from jax import lax
from jax.experimental import pallas as pl
from jax.experimental.pallas import tpu as pltpu
from jax.experimental.pallas import tpu_sc as plsc

VOCAB = 1000
VOCAB_PAD = 1008
BATCH = 16384
NUM_CORES = 1
NUM_SUBCORES = 16
LANES = 16
NUM_WORKERS = NUM_CORES * NUM_SUBCORES
B_PER_W = BATCH // NUM_WORKERS

_mesh = plsc.VectorSubcoreMesh(core_axis_name="c", subcore_axis_name="s",
                               num_cores=NUM_CORES)


@functools.partial(
    pl.kernel,
    mesh=_mesh,
    out_type=jax.ShapeDtypeStruct((BATCH,), jnp.float32),
    scratch_types=[
        pltpu.VMEM((VOCAB,), jnp.float32),
        pltpu.VMEM((B_PER_W,), jnp.int32),
        pltpu.VMEM((B_PER_W,), jnp.float32),
        pltpu.SemaphoreType.DMA,
        pltpu.SemaphoreType.DMA,
    ],
    compiler_params=pltpu.CompilerParams(needs_layout_passes=False),
)
def _ig_gather_sigmoid(genes_hbm, ig_hbm, out_hbm, tab_v, idx_v, out_v,
                       sem_in, sem_out):
    wid = lax.axis_index("s") * NUM_CORES + lax.axis_index("c")
    base = wid * B_PER_W
    tab_cp = pltpu.async_copy(ig_hbm, tab_v, sem_in)
    idx_cp = pltpu.async_copy(genes_hbm.at[pl.ds(base, B_PER_W)], idx_v,
                              sem_in)
    tab_cp.wait()
    idx_cp.wait()

    def step(off):
        idx = idx_v[pl.ds(off, LANES)]
        g = plsc.load_gather(tab_v, [idx])
        out_v[pl.ds(off, LANES)] = 1.0 / (1.0 + jnp.exp(-g))

    plsc.parallel_loop(0, B_PER_W, step=LANES, unroll=2)(step)
    out_cp = pltpu.async_copy(out_v, out_hbm.at[pl.ds(base, B_PER_W)],
                              sem_out)
    out_cp.wait()


def kernel(current_genes, ig):
    return _ig_gather_sigmoid(current_genes.astype(jnp.int32), ig)


# speedup vs baseline: 5.1450x; 1.0045x over previous
import functools

import jax
import jax.numpy as jnp
from jax import lax
from jax.experimental import pallas as pl
from jax.experimental.pallas import tpu as pltpu
from jax.experimental.pallas import tpu_sc as plsc

VOCAB = 1000
VOCAB_PAD = 1008
BATCH = 16384
NUM_CORES = 1
NUM_SUBCORES = 16
LANES = 16
NUM_WORKERS = NUM_CORES * NUM_SUBCORES
B_PER_W = BATCH // NUM_WORKERS

_mesh = plsc.VectorSubcoreMesh(core_axis_name="c", subcore_axis_name="s",
                               num_cores=NUM_CORES)


@functools.partial(
    pl.kernel,
    mesh=_mesh,
    out_type=jax.ShapeDtypeStruct((BATCH,), jnp.float32),
    scratch_types=[
        pltpu.VMEM((VOCAB,), jnp.float32),
        pltpu.VMEM((B_PER_W,), jnp.int32),
        pltpu.VMEM((B_PER_W,), jnp.float32),
        pltpu.SemaphoreType.DMA,
        pltpu.SemaphoreType.DMA,
    ],
    compiler_params=pltpu.CompilerParams(needs_layout_passes=False),
)
def _ig_gather_sigmoid(genes_hbm, ig_hbm, out_hbm, tab_v, idx_v, out_v,
                       sem_in, sem_out):
    wid = lax.axis_index("s") * NUM_CORES + lax.axis_index("c")
    base = wid * B_PER_W
    tab_cp = pltpu.async_copy(ig_hbm, tab_v, sem_in)
    idx_cp = pltpu.async_copy(genes_hbm.at[pl.ds(base, B_PER_W)], idx_v,
                              sem_in)
    tab_cp.wait()
    idx_cp.wait()

    def step(off):
        idx = idx_v[pl.ds(off, LANES)]
        g = plsc.load_gather(tab_v, [idx])
        out_v[pl.ds(off, LANES)] = 1.0 / (1.0 + jnp.exp(-g))

    plsc.parallel_loop(0, B_PER_W, step=LANES, unroll=3)(step)
    out_cp = pltpu.async_copy(out_v, out_hbm.at[pl.ds(base, B_PER_W)],
                              sem_out)
    out_cp.wait()


def kernel(current_genes, ig):
    return _ig_gather_sigmoid(current_genes.astype(jnp.int32), ig)
